# baseline (speedup 1.0000x reference)
import functools

import jax
import jax.numpy as jnp
from jax import lax
from jax.experimental import pallas as pl
from jax.experimental.pallas import tpu as pltpu
from jax.experimental.pallas import tpu_sc as plsc

N = 19
E = 342
D = 512
_NCHUNK = (E + 15) // 16
_EPAD = _NCHUNK * 16


def _pair_logits(xl, xr, a2d):
    f32 = jnp.float32
    u = jax.lax.dot_general(a2d, xl, (((1,), (1,)), ((), ())),
                            preferred_element_type=f32)
    v = jax.lax.dot_general(a2d, xr, (((1,), (1,)), ((), ())),
                            preferred_element_type=f32)
    rows = []
    for s in range(N):
        az = jnp.abs(xl[s:s + 1, :] + xr)
        w = jax.lax.dot_general(a2d, az, (((1,), (1,)), ((), ())),
                                preferred_element_type=f32)
        rows.append(0.6 * (u[0:1, s:s + 1] + v) + 0.4 * w)
    return jnp.concatenate(rows, axis=0)


def _tc_fused(x_ref, ei_ref, mmse_ref, wl1_ref, wr1_ref, a1_ref, b1_ref,
              wl2_ref, wr2_ref, a2_ref, b2_ref, wm_ref, bm_ref,
              comp_ref, alpha2d_ref):
    f32 = jnp.float32
    src = ei_ref[0:1, :]
    dst = ei_ref[1:2, :]
    iota_ne = jax.lax.broadcasted_iota(jnp.int32, (N, E), 0)
    s_oh = (iota_ne == src).astype(f32)
    d_oh = (iota_ne == dst).astype(f32)
    c2 = jax.lax.dot_general(s_oh, d_oh, (((1,), (1,)), ((), ())),
                             preferred_element_type=f32)
    has = c2 > 0.0

    def gatv2(h, wl, wr, a, b):
        xl = jnp.dot(h, wl, preferred_element_type=f32)
        xr = jnp.dot(h, wr, preferred_element_type=f32)
        e2 = _pair_logits(xl, xr, a)
        m = jnp.max(jnp.where(has, e2, -1e30), axis=0, keepdims=True)
        ex = jnp.where(has, jnp.exp(e2 - m), 0.0)
        ssum = jnp.sum(c2 * ex, axis=0, keepdims=True)
        alpha = ex / (ssum + 1e-16)
        wmat = c2 * alpha
        out = jax.lax.dot_general(wmat, xl, (((0,), (0,)), ((), ())),
                                  preferred_element_type=f32)
        return out + b, alpha

    h1, alpha1 = gatv2(x_ref[...], wl1_ref[...], wr1_ref[...],
                       a1_ref[...].reshape(1, D), b1_ref[...].reshape(1, D))
    alpha2d_ref[...] = alpha1
    h2, _ = gatv2(h1, wl2_ref[...], wr2_ref[...],
                  a2_ref[...].reshape(1, D), b2_ref[...].reshape(1, D))
    gf = (h2 + mmse_ref[...].reshape(1, 1) * wm_ref[...]
          + bm_ref[...].reshape(1, D))
    dec = jax.lax.dot_general(gf, gf, (((1,), (1,)), ((), ())),
                              preferred_element_type=f32)
    comp_ref[...] = jax.nn.sigmoid(dec)


def _sc_alpha_gather(alpha2d, edge_index):
    mesh = plsc.VectorSubcoreMesh(core_axis_name="c", subcore_axis_name="s")
    offs = [i * 16 for i in range(E // 16)] + [E - 16]

    @functools.partial(
        pl.kernel, mesh=mesh,
        compiler_params=pltpu.CompilerParams(needs_layout_passes=False),
        out_type=jax.ShapeDtypeStruct((E,), jnp.float32),
        scratch_types=[
            pltpu.VMEM((N, N), jnp.float32),
            pltpu.VMEM((N * N,), jnp.float32),
            pltpu.VMEM((2, E), jnp.int32),
            pltpu.VMEM((E,), jnp.float32),
        ],
    )
    def k(alpha_hbm, ei_hbm, out_hbm, al_v, alf_v, ei_v, res_v):
        wid = lax.axis_index("s") * 2 + lax.axis_index("c")

        @pl.when(wid == 0)
        def _work():
            pltpu.sync_copy(alpha_hbm, al_v)
            pltpu.sync_copy(ei_hbm, ei_v)
            for r in range(N):
                alf_v[pl.ds(19 * r, 16)] = al_v[r, pl.ds(0, 16)]
                alf_v[pl.ds(19 * r + 3, 16)] = al_v[r, pl.ds(3, 16)]
            for off in offs:
                srci = ei_v[0, pl.ds(off, 16)]
                dsti = ei_v[1, pl.ds(off, 16)]
                p = srci * N + dsti
                res_v[pl.ds(off, 16)] = plsc.load_gather(alf_v, [p])
            pltpu.sync_copy(res_v, out_hbm)

    return k(alpha2d, edge_index)


def kernel(x, edge_index, mmse, Wl1, Wr1, a1, b1, Wl2, Wr2, a2, b2, Wm, bm,
           W11, b11, W12, b12, W21, b21, W22, b22, Wp, bp):
    f32 = jnp.float32
    T = x.shape[1]
    compressed, alpha2d = pl.pallas_call(
        _tc_fused,
        in_specs=[pl.BlockSpec((19, T), lambda: (0, 0)),
                  pl.BlockSpec((2, E), lambda: (0, 0)),
                  pl.BlockSpec((1,), lambda: (0,)),
                  pl.BlockSpec((T, D), lambda: (0, 0)),
                  pl.BlockSpec((T, D), lambda: (0, 0)),
                  pl.BlockSpec((D,), lambda: (0,)),
                  pl.BlockSpec((D,), lambda: (0,)),
                  pl.BlockSpec((D, D), lambda: (0, 0)),
                  pl.BlockSpec((D, D), lambda: (0, 0)),
                  pl.BlockSpec((D,), lambda: (0,)),
                  pl.BlockSpec((D,), lambda: (0,)),
                  pl.BlockSpec((1, D), lambda: (0, 0)),
                  pl.BlockSpec((D,), lambda: (0,))],
        out_shape=[
            jax.ShapeDtypeStruct((N, N), f32),
            jax.ShapeDtypeStruct((N, N), f32),
        ],
    )(x, edge_index, mmse,
      Wl1, Wr1, a1, b1, Wl2, Wr2, a2, b2, Wm, bm)
    alpha1 = _sc_alpha_gather(alpha2d, edge_index)
    return compressed, alpha1


# speedup vs baseline: 3.7981x; 3.7981x over previous
import jax
import jax.numpy as jnp
from jax.experimental import pallas as pl

N = 19
E = 342
D = 512
_HI = jax.lax.Precision.DEFAULT


def _pair_logits(xl, xr, a2d):
    f32 = jnp.float32
    u = jax.lax.dot_general(a2d, xl, (((1,), (1,)), ((), ())),
                            precision=_HI, preferred_element_type=f32)
    v = jax.lax.dot_general(a2d, xr, (((1,), (1,)), ((), ())),
                            precision=_HI, preferred_element_type=f32)
    rows = []
    for s in range(N):
        az = jnp.abs(xl[s:s + 1, :] + xr)
        w = jax.lax.dot_general(a2d, az, (((1,), (1,)), ((), ())),
                                precision=_HI, preferred_element_type=f32)
        rows.append(0.6 * (u[0:1, s:s + 1] + v) + 0.4 * w)
    return jnp.concatenate(rows, axis=0)


def _fused(x_ref, ei_ref, mmse_ref, wl1_ref, wr1_ref, a1_ref, b1_ref,
           wl2_ref, wr2_ref, a2_ref, b2_ref, wm_ref, bm_ref,
           comp_ref, alpha_ref):
    f32 = jnp.float32
    src = ei_ref[0:1, :]
    dst = ei_ref[1:2, :]
    iota_ne = jax.lax.broadcasted_iota(jnp.int32, (N, E), 0)
    s_oh = (iota_ne == src).astype(f32)
    d_oh = (iota_ne == dst).astype(f32)
    c2 = jax.lax.dot_general(s_oh, d_oh, (((1,), (1,)), ((), ())),
                             preferred_element_type=f32)
    has = c2 > 0.0

    def gatv2(h, wl, wr, a, b):
        xl = jnp.dot(h, wl, precision=_HI, preferred_element_type=f32)
        xr = jnp.dot(h, wr, precision=_HI, preferred_element_type=f32)
        e2 = _pair_logits(xl, xr, a)
        m = jnp.max(jnp.where(has, e2, -1e30), axis=0, keepdims=True)
        ex = jnp.where(has, jnp.exp(e2 - m), 0.0)
        ssum = jnp.sum(c2 * ex, axis=0, keepdims=True)
        alpha = ex / (ssum + 1e-16)
        wmat = c2 * alpha
        out = jax.lax.dot_general(wmat, xl, (((0,), (0,)), ((), ())),
                                  precision=_HI, preferred_element_type=f32)
        return out + b, alpha

    h1, alpha1 = gatv2(x_ref[...], wl1_ref[...], wr1_ref[...],
                       a1_ref[...].reshape(1, D), b1_ref[...].reshape(1, D))
    h2, _ = gatv2(h1, wl2_ref[...], wr2_ref[...],
                  a2_ref[...].reshape(1, D), b2_ref[...].reshape(1, D))
    gf = h2 + mmse_ref[...].reshape(1, 1) * wm_ref[...] + bm_ref[...].reshape(1, D)
    dec = jax.lax.dot_general(gf, gf, (((1,), (1,)), ((), ())),
                              precision=_HI, preferred_element_type=f32)
    comp_ref[...] = jax.nn.sigmoid(dec)
    u = jax.lax.dot_general(alpha1, d_oh, (((1,), (0,)), ((), ())),
                            precision=_HI, preferred_element_type=f32)
    alpha_ref[...] = jnp.sum(s_oh * u, axis=0)


def kernel(x, edge_index, mmse, Wl1, Wr1, a1, b1, Wl2, Wr2, a2, b2, Wm, bm,
           W11, b11, W12, b12, W21, b21, W22, b22, Wp, bp):
    compressed, alpha_1d = pl.pallas_call(
        _fused,
        out_shape=[
            jax.ShapeDtypeStruct((N, N), jnp.float32),
            jax.ShapeDtypeStruct((E,), jnp.float32),
        ],
    )(x, edge_index, mmse,
      Wl1, Wr1, a1, b1, Wl2, Wr2, a2, b2, Wm, bm)
    return compressed, alpha_1d
